# baseline (speedup 1.0000x reference)
import functools

import jax
import jax.numpy as jnp
from jax import lax
from jax.experimental import pallas as pl
from jax.experimental.pallas import tpu as pltpu
from jax.experimental.pallas import tpu_sc as plsc

K = 16
TM = 256

_SC_CORES = 2
_SC_SUBCORES = 16
_SC_WORKERS = _SC_CORES * _SC_SUBCORES
_QCHUNK = 128


_CL = 128


def _topk_tc_body(q_ref, xt_ref, tinv2_ref, w_ref, idx_ref):
    n = xt_ref.shape[2]
    q = q_ref[0]
    x = xt_ref[0]
    d0 = q[:, 0:1] - x[0][None, :]
    d1 = q[:, 1:2] - x[1][None, :]
    d2c = q[:, 2:3] - x[2][None, :]
    s = (d0 * d0 + d1 * d1) + d2c * d2c
    big = jnp.float32(3.0e38)
    nfill = jnp.int32(n)
    iota = lax.broadcasted_iota(jnp.int32, s.shape, 1)

    vals = []
    idxs = []
    for _ in range(K):
        v = jnp.min(s, axis=1)
        iv = jnp.argmin(s, axis=1).astype(jnp.int32)
        s = jnp.where(iota == iv[:, None], big, s)
        vals.append(v)
        idxs.append(iv)
    tinv2 = tinv2_ref[0]
    v0 = vals[0]
    es = [jnp.exp((v0 - v) * tinv2) for v in vals]
    z = es[0]
    for e in es[1:]:
        z = z + e
    zin = 1.0 / z
    for k in range(K):
        w_ref[0, :, k] = es[k] * zin
        idx_ref[0, :, k] = idxs[k]


def _topk_tc(sample_xyz, xt, tinv2):
    b, m, _ = sample_xyz.shape
    n = xt.shape[2]
    grid = (b, m // TM)
    return pl.pallas_call(
        _topk_tc_body,
        grid=grid,
        compiler_params=pltpu.CompilerParams(
            dimension_semantics=("parallel", "parallel"),
        ),
        in_specs=[
            pl.BlockSpec((1, TM, 3), lambda i, j: (i, j, 0)),
            pl.BlockSpec((1, 3, n), lambda i, j: (i, 0, 0)),
            pl.BlockSpec(memory_space=pltpu.SMEM),
        ],
        out_specs=[
            pl.BlockSpec((1, TM, K), lambda i, j: (i, j, 0)),
            pl.BlockSpec((1, TM, K), lambda i, j: (i, j, 0)),
        ],
        out_shape=[
            jax.ShapeDtypeStruct((b, m, K), jnp.float32),
            jax.ShapeDtypeStruct((b, m, K), jnp.int32),
        ],
    )(sample_xyz, xt, tinv2)


def _sc_aggregate(xflat, wt, idxt, n, nb):
    chunks_per_b = _SC_WORKERS // nb
    mesh = plsc.VectorSubcoreMesh(core_axis_name="c", subcore_axis_name="s")

    @functools.partial(
        pl.kernel,
        mesh=mesh,
        compiler_params=pltpu.CompilerParams(needs_layout_passes=False),
        out_type=jax.ShapeDtypeStruct((_SC_WORKERS, 3, _QCHUNK), jnp.float32),
        scratch_types=[
            pltpu.VMEM((n,), jnp.float32),
            pltpu.VMEM((n,), jnp.float32),
            pltpu.VMEM((n,), jnp.float32),
            pltpu.VMEM((K, _QCHUNK), jnp.float32),
            pltpu.VMEM((K, _QCHUNK), jnp.int32),
            pltpu.VMEM((3, _QCHUNK), jnp.float32),
        ],
    )
    def body(xt_hbm, w_hbm, idx_hbm, out_hbm, xv, yv, zv, wv, iv, ov):
        wid = lax.axis_index("s") * _SC_CORES + lax.axis_index("c")
        bb = wid // chunks_per_b
        base = bb * (3 * n)
        pltpu.sync_copy(xt_hbm.at[pl.ds(base, n)], xv)
        pltpu.sync_copy(xt_hbm.at[pl.ds(base + n, n)], yv)
        pltpu.sync_copy(xt_hbm.at[pl.ds(base + 2 * n, n)], zv)
        pltpu.sync_copy(w_hbm.at[wid], wv)
        pltpu.sync_copy(idx_hbm.at[wid], iv)
        for g in range(_QCHUNK // 16):
            ax = jnp.zeros((16,), jnp.float32)
            ay = jnp.zeros((16,), jnp.float32)
            az = jnp.zeros((16,), jnp.float32)
            for k in range(K):
                ii = iv[k, pl.ds(g * 16, 16)]
                ww = wv[k, pl.ds(g * 16, 16)]
                ax = ax + ww * plsc.load_gather(xv, [ii])
                ay = ay + ww * plsc.load_gather(yv, [ii])
                az = az + ww * plsc.load_gather(zv, [ii])
            ov[0, pl.ds(g * 16, 16)] = ax
            ov[1, pl.ds(g * 16, 16)] = ay
            ov[2, pl.ds(g * 16, 16)] = az
        pltpu.sync_copy(ov, out_hbm.at[wid])

    return body(xflat, wt, idxt)


def kernel(xyz, sample_xyz, temp):
    b, n, _ = xyz.shape
    m = sample_xyz.shape[1]
    xt = jnp.transpose(xyz, (0, 2, 1))
    tinv2 = (1.0 / (temp * temp)).reshape(1).astype(jnp.float32)

    w, idx = _topk_tc(sample_xyz, xt, tinv2)

    chunks_per_b = _SC_WORKERS // b
    wt = (
        w.reshape(b, chunks_per_b, _QCHUNK, K)
        .transpose(0, 1, 3, 2)
        .reshape(_SC_WORKERS, K, _QCHUNK)
    )
    idxt = (
        idx.reshape(b, chunks_per_b, _QCHUNK, K)
        .transpose(0, 1, 3, 2)
        .reshape(_SC_WORKERS, K, _QCHUNK)
    )
    out = _sc_aggregate(xt.reshape(b * 3 * n), wt, idxt, n, b)
    proj = (
        out.reshape(b, chunks_per_b, 3, _QCHUNK)
        .transpose(0, 1, 3, 2)
        .reshape(b, m, 3)
    )
    return (proj, temp)


# speedup vs baseline: 1.2001x; 1.2001x over previous
import functools

import jax
import jax.numpy as jnp
from jax import lax
from jax.experimental import pallas as pl
from jax.experimental.pallas import tpu as pltpu
from jax.experimental.pallas import tpu_sc as plsc

K = 16
TM = 256

_SC_CORES = 2
_SC_SUBCORES = 16
_SC_WORKERS = _SC_CORES * _SC_SUBCORES
_QCHUNK = 128


_CL = 128


def _topk_tc_body(q_ref, xt_ref, tinv2_ref, w_ref, idx_ref):
    n = xt_ref.shape[2]
    q = q_ref[0]
    x = xt_ref[0]
    d0 = q[:, 0:1] - x[0][None, :]
    d1 = q[:, 1:2] - x[1][None, :]
    d2c = q[:, 2:3] - x[2][None, :]
    s = (d0 * d0 + d1 * d1) + d2c * d2c
    big = jnp.float32(3.0e38)
    nfill = jnp.int32(n)
    iota = lax.broadcasted_iota(jnp.int32, s.shape, 1)

    vals = []
    idxs = []
    for _ in range(K):
        v = jnp.min(s, axis=1)
        iv = jnp.min(jnp.where(s == v[:, None], iota, nfill), axis=1)
        s = jnp.where(iota == iv[:, None], big, s)
        vals.append(v)
        idxs.append(iv)
    tinv2 = tinv2_ref[0]
    v0 = vals[0]
    es = [jnp.exp((v0 - v) * tinv2) for v in vals]
    z = es[0]
    for e in es[1:]:
        z = z + e
    zin = 1.0 / z
    for k in range(K):
        w_ref[0, :, k] = es[k] * zin
        idx_ref[0, :, k] = idxs[k]


def _topk_tc(sample_xyz, xt, tinv2):
    b, m, _ = sample_xyz.shape
    n = xt.shape[2]
    grid = (b, m // TM)
    return pl.pallas_call(
        _topk_tc_body,
        grid=grid,
        compiler_params=pltpu.CompilerParams(
            dimension_semantics=("parallel", "parallel"),
        ),
        in_specs=[
            pl.BlockSpec((1, TM, 3), lambda i, j: (i, j, 0)),
            pl.BlockSpec((1, 3, n), lambda i, j: (i, 0, 0)),
            pl.BlockSpec(memory_space=pltpu.SMEM),
        ],
        out_specs=[
            pl.BlockSpec((1, TM, K), lambda i, j: (i, j, 0)),
            pl.BlockSpec((1, TM, K), lambda i, j: (i, j, 0)),
        ],
        out_shape=[
            jax.ShapeDtypeStruct((b, m, K), jnp.float32),
            jax.ShapeDtypeStruct((b, m, K), jnp.int32),
        ],
    )(sample_xyz, xt, tinv2)


def _sc_aggregate(xflat, wt, idxt, n, nb):
    chunks_per_b = _SC_WORKERS // nb
    mesh = plsc.VectorSubcoreMesh(core_axis_name="c", subcore_axis_name="s")

    @functools.partial(
        pl.kernel,
        mesh=mesh,
        compiler_params=pltpu.CompilerParams(needs_layout_passes=False),
        out_type=jax.ShapeDtypeStruct((_SC_WORKERS, 3, _QCHUNK), jnp.float32),
        scratch_types=[
            pltpu.VMEM((n,), jnp.float32),
            pltpu.VMEM((n,), jnp.float32),
            pltpu.VMEM((n,), jnp.float32),
            pltpu.VMEM((K, _QCHUNK), jnp.float32),
            pltpu.VMEM((K, _QCHUNK), jnp.int32),
            pltpu.VMEM((3, _QCHUNK), jnp.float32),
        ],
    )
    def body(xt_hbm, w_hbm, idx_hbm, out_hbm, xv, yv, zv, wv, iv, ov):
        wid = lax.axis_index("s") * _SC_CORES + lax.axis_index("c")
        bb = wid // chunks_per_b
        base = bb * (3 * n)
        pltpu.sync_copy(xt_hbm.at[pl.ds(base, n)], xv)
        pltpu.sync_copy(xt_hbm.at[pl.ds(base + n, n)], yv)
        pltpu.sync_copy(xt_hbm.at[pl.ds(base + 2 * n, n)], zv)
        pltpu.sync_copy(w_hbm.at[wid], wv)
        pltpu.sync_copy(idx_hbm.at[wid], iv)
        for g in range(_QCHUNK // 16):
            ax = jnp.zeros((16,), jnp.float32)
            ay = jnp.zeros((16,), jnp.float32)
            az = jnp.zeros((16,), jnp.float32)
            for k in range(K):
                ii = iv[k, pl.ds(g * 16, 16)]
                ww = wv[k, pl.ds(g * 16, 16)]
                ax = ax + ww * plsc.load_gather(xv, [ii])
                ay = ay + ww * plsc.load_gather(yv, [ii])
                az = az + ww * plsc.load_gather(zv, [ii])
            ov[0, pl.ds(g * 16, 16)] = ax
            ov[1, pl.ds(g * 16, 16)] = ay
            ov[2, pl.ds(g * 16, 16)] = az
        pltpu.sync_copy(ov, out_hbm.at[wid])

    return body(xflat, wt, idxt)


def kernel(xyz, sample_xyz, temp):
    b, n, _ = xyz.shape
    m = sample_xyz.shape[1]
    xt = jnp.transpose(xyz, (0, 2, 1))
    tinv2 = (1.0 / (temp * temp)).reshape(1).astype(jnp.float32)

    w, idx = _topk_tc(sample_xyz, xt, tinv2)

    chunks_per_b = _SC_WORKERS // b
    wt = (
        w.reshape(b, chunks_per_b, _QCHUNK, K)
        .transpose(0, 1, 3, 2)
        .reshape(_SC_WORKERS, K, _QCHUNK)
    )
    idxt = (
        idx.reshape(b, chunks_per_b, _QCHUNK, K)
        .transpose(0, 1, 3, 2)
        .reshape(_SC_WORKERS, K, _QCHUNK)
    )
    out = _sc_aggregate(xt.reshape(b * 3 * n), wt, idxt, n, b)
    proj = (
        out.reshape(b, chunks_per_b, 3, _QCHUNK)
        .transpose(0, 1, 3, 2)
        .reshape(b, m, 3)
    )
    return (proj, temp)


# speedup vs baseline: 1.3008x; 1.0840x over previous
import functools

import jax
import jax.numpy as jnp
from jax import lax
from jax.experimental import pallas as pl
from jax.experimental.pallas import tpu as pltpu
from jax.experimental.pallas import tpu_sc as plsc

K = 16
TM = 512

_SC_CORES = 2
_SC_SUBCORES = 16
_SC_WORKERS = _SC_CORES * _SC_SUBCORES
_QCHUNK = 128


_CL = 128


def _topk_tc_body(q_ref, xt_ref, tinv2_ref, w_ref, idx_ref):
    n = xt_ref.shape[2]
    q = q_ref[0]
    x = xt_ref[0]
    d0 = q[:, 0:1] - x[0][None, :]
    d1 = q[:, 1:2] - x[1][None, :]
    d2c = q[:, 2:3] - x[2][None, :]
    s = (d0 * d0 + d1 * d1) + d2c * d2c
    big = jnp.float32(3.0e38)
    nfill = jnp.int32(n)
    iota = lax.broadcasted_iota(jnp.int32, s.shape, 1)

    vals = []
    idxs = []
    for _ in range(K):
        v = jnp.min(s, axis=1)
        iv = jnp.min(jnp.where(s == v[:, None], iota, nfill), axis=1)
        s = jnp.where(iota == iv[:, None], big, s)
        vals.append(v)
        idxs.append(iv)
    tinv2 = tinv2_ref[0]
    v0 = vals[0]
    es = [jnp.exp((v0 - v) * tinv2) for v in vals]
    z = es[0]
    for e in es[1:]:
        z = z + e
    zin = 1.0 / z
    for k in range(K):
        w_ref[0, :, k] = es[k] * zin
        idx_ref[0, :, k] = idxs[k]


def _topk_tc(sample_xyz, xt, tinv2):
    b, m, _ = sample_xyz.shape
    n = xt.shape[2]
    grid = (b, m // TM)
    return pl.pallas_call(
        _topk_tc_body,
        grid=grid,
        compiler_params=pltpu.CompilerParams(
            dimension_semantics=("parallel", "parallel"),
        ),
        in_specs=[
            pl.BlockSpec((1, TM, 3), lambda i, j: (i, j, 0)),
            pl.BlockSpec((1, 3, n), lambda i, j: (i, 0, 0)),
            pl.BlockSpec(memory_space=pltpu.SMEM),
        ],
        out_specs=[
            pl.BlockSpec((1, TM, K), lambda i, j: (i, j, 0)),
            pl.BlockSpec((1, TM, K), lambda i, j: (i, j, 0)),
        ],
        out_shape=[
            jax.ShapeDtypeStruct((b, m, K), jnp.float32),
            jax.ShapeDtypeStruct((b, m, K), jnp.int32),
        ],
    )(sample_xyz, xt, tinv2)


def _sc_aggregate(xflat, wt, idxt, n, nb):
    chunks_per_b = _SC_WORKERS // nb
    mesh = plsc.VectorSubcoreMesh(core_axis_name="c", subcore_axis_name="s")

    @functools.partial(
        pl.kernel,
        mesh=mesh,
        compiler_params=pltpu.CompilerParams(needs_layout_passes=False),
        out_type=jax.ShapeDtypeStruct((_SC_WORKERS, 3, _QCHUNK), jnp.float32),
        scratch_types=[
            pltpu.VMEM((n,), jnp.float32),
            pltpu.VMEM((n,), jnp.float32),
            pltpu.VMEM((n,), jnp.float32),
            pltpu.VMEM((K, _QCHUNK), jnp.float32),
            pltpu.VMEM((K, _QCHUNK), jnp.int32),
            pltpu.VMEM((3, _QCHUNK), jnp.float32),
        ],
    )
    def body(xt_hbm, w_hbm, idx_hbm, out_hbm, xv, yv, zv, wv, iv, ov):
        wid = lax.axis_index("s") * _SC_CORES + lax.axis_index("c")
        bb = wid // chunks_per_b
        base = bb * (3 * n)
        pltpu.sync_copy(xt_hbm.at[pl.ds(base, n)], xv)
        pltpu.sync_copy(xt_hbm.at[pl.ds(base + n, n)], yv)
        pltpu.sync_copy(xt_hbm.at[pl.ds(base + 2 * n, n)], zv)
        pltpu.sync_copy(w_hbm.at[wid], wv)
        pltpu.sync_copy(idx_hbm.at[wid], iv)
        for g in range(_QCHUNK // 16):
            ax = jnp.zeros((16,), jnp.float32)
            ay = jnp.zeros((16,), jnp.float32)
            az = jnp.zeros((16,), jnp.float32)
            for k in range(K):
                ii = iv[k, pl.ds(g * 16, 16)]
                ww = wv[k, pl.ds(g * 16, 16)]
                ax = ax + ww * plsc.load_gather(xv, [ii])
                ay = ay + ww * plsc.load_gather(yv, [ii])
                az = az + ww * plsc.load_gather(zv, [ii])
            ov[0, pl.ds(g * 16, 16)] = ax
            ov[1, pl.ds(g * 16, 16)] = ay
            ov[2, pl.ds(g * 16, 16)] = az
        pltpu.sync_copy(ov, out_hbm.at[wid])

    return body(xflat, wt, idxt)


def kernel(xyz, sample_xyz, temp):
    b, n, _ = xyz.shape
    m = sample_xyz.shape[1]
    xt = jnp.transpose(xyz, (0, 2, 1))
    tinv2 = (1.0 / (temp * temp)).reshape(1).astype(jnp.float32)

    w, idx = _topk_tc(sample_xyz, xt, tinv2)

    chunks_per_b = _SC_WORKERS // b
    wt = (
        w.reshape(b, chunks_per_b, _QCHUNK, K)
        .transpose(0, 1, 3, 2)
        .reshape(_SC_WORKERS, K, _QCHUNK)
    )
    idxt = (
        idx.reshape(b, chunks_per_b, _QCHUNK, K)
        .transpose(0, 1, 3, 2)
        .reshape(_SC_WORKERS, K, _QCHUNK)
    )
    out = _sc_aggregate(xt.reshape(b * 3 * n), wt, idxt, n, b)
    proj = (
        out.reshape(b, chunks_per_b, 3, _QCHUNK)
        .transpose(0, 1, 3, 2)
        .reshape(b, m, 3)
    )
    return (proj, temp)
